# Initial kernel scaffold; baseline (speedup 1.0000x reference)
import jax
import jax.numpy as jnp
from jax.experimental import pallas as pl


def kernel(feat_0, feat_1, mem_0, mem_1):
    raise NotImplementedError("write your pallas kernel here")


# speedup vs baseline: 1.3518x; 1.3518x over previous
import functools

import jax
import jax.numpy as jnp
from jax import lax
from jax.experimental import pallas as pl
from jax.experimental.pallas import tpu as pltpu
from jax.experimental.pallas import tpu_sc as plsc

B, D, N = 1024, 128, 100000
K = 10
TEMP = 3.0
W = 2048
NB = (N + W - 1) // W
RB = 256
NRB = B // RB
NEG = -3.0e38
IMAX = 2**31 - 1
ZC = float((N - K) * 1e-8 * 18.420680743952367)

NW = 32
ROWS_PW = (2 * B) // NW
VALS_PW = ROWS_PW * K
VPAD = VALS_PW + 16


def _tc_body(feat_ref, mem0_ref, mem1_ref, sims_ref, pv_ref, pi_ref,
             rv, ri, mn):
    l = pl.program_id(0)
    j = pl.program_id(1)
    b = pl.program_id(2)

    @pl.when(j == 0)
    def _():
        rv[pl.ds(b * RB, RB), :] = jnp.full((RB, 2 * K), NEG, jnp.float32)
        ri[pl.ds(b * RB, RB), :] = jnp.full((RB, 2 * K), IMAX, jnp.int32)

    @pl.when(b == 0)
    def _():
        m = jnp.where(l == 0, mem0_ref[...], mem1_ref[...])
        nrm = jnp.sqrt(jnp.sum(m * m, axis=1, keepdims=True))
        mn[...] = m / jnp.maximum(nrm, 1e-12)

    f = feat_ref[0]
    fn = f / jnp.maximum(
        jnp.sqrt(jnp.sum(f * f, axis=1, keepdims=True)), 1e-12)
    s = lax.dot_general(fn, mn[...], (((1,), (1,)), ((), ())),
                        preferred_element_type=jnp.float32)
    col = j * W + lax.broadcasted_iota(jnp.int32, (RB, W), 1)
    s = jnp.where(col < N, s, NEG)
    sims_ref[0, :, :] = s

    lanes = lax.broadcasted_iota(jnp.int32, (RB, 2 * K), 1)

    def ext_body(t, carry):
        sc, accv, acci = carry
        m = jnp.max(sc, axis=1, keepdims=True)
        cand = jnp.where(sc == m, col, IMAX)
        am = jnp.min(cand, axis=1, keepdims=True)
        sc = jnp.where(cand == am, NEG, sc)
        accv = jnp.where(lanes == K + t, m, accv)
        acci = jnp.where(lanes == K + t, am, acci)
        return sc, accv, acci

    accv0 = rv[pl.ds(b * RB, RB), :]
    acci0 = ri[pl.ds(b * RB, RB), :]
    _, accv, acci = lax.fori_loop(0, K, ext_body, (s, accv0, acci0))

    def mrg_body(t, carry):
        a, ai, ov, oi = carry
        m = jnp.max(a, axis=1, keepdims=True)
        cand = jnp.where(a == m, ai, IMAX)
        am = jnp.min(cand, axis=1, keepdims=True)
        a = jnp.where(cand == am, NEG, a)
        ov = jnp.where(lanes == t, m, ov)
        oi = jnp.where(lanes == t, am, oi)
        return a, ai, ov, oi

    ovi = jnp.full((RB, 2 * K), NEG, jnp.float32)
    oii = jnp.full((RB, 2 * K), IMAX, jnp.int32)
    _, _, ov, oi = lax.fori_loop(0, K, mrg_body, (accv, acci, ovi, oii))
    rv[pl.ds(b * RB, RB), :] = ov
    ri[pl.ds(b * RB, RB), :] = oi

    @pl.when(j == NB - 1)
    def _():
        v16 = ov[:, 0:16]
        v0 = ov[:, 0:1]
        e = jnp.exp((v16 - v0) / TEMP)
        z = jnp.sum(e, axis=1, keepdims=True)
        pv_ref[0, :, :] = e / z
        pi_ref[0, :, :] = oi[:, 0:16]


def _wt_body(pv_ref, w_ref):
    p = pv_ref[0]
    lane = lax.broadcasted_iota(jnp.int32, (B, 16), 1)
    term = jnp.where(lane < K, p * jnp.log(jnp.maximum(p, 1e-30)), 0.0)
    ent = -jnp.sum(term, axis=1, keepdims=True) + ZC
    w = jnp.exp(-ent)
    mw = jnp.sum(w) / B
    w_ref[0, :, :] = jnp.broadcast_to(w / (mw + 1e-8), (B, 128))


def _sc_body(idx_hbm, val_hbm, out_hbm, rowbuf, idxbuf, valbuf):
    wid = lax.axis_index("s") * 2 + lax.axis_index("c")
    pltpu.sync_copy(idx_hbm.at[wid], idxbuf)
    pltpu.sync_copy(val_hbm.at[wid], valbuf)

    zero16 = jnp.zeros((16,), jnp.float32)

    def zbody(i, c):
        rowbuf[pl.ds(i * 16, 16)] = zero16
        return c

    lax.fori_loop(0, N // 16, zbody, 0)

    kmask = lax.iota(jnp.int32, 16) < K

    def rbody(r, c):
        idxv = idxbuf[pl.ds(r * K, 16)]
        valv = valbuf[pl.ds(r * K, 16)]
        plsc.store_scatter(rowbuf, [idxv], valv, mask=kmask)
        base = (wid * ROWS_PW + r) * N
        pltpu.sync_copy(rowbuf, out_hbm.at[pl.ds(base, N)])
        plsc.store_scatter(rowbuf, [idxv], zero16, mask=kmask)
        return c

    lax.fori_loop(0, ROWS_PW, rbody, 0)


def kernel(feat_0, feat_1, mem_0, mem_1):
    feats = jnp.stack([feat_0, feat_1])

    sims, pvals, pidx = pl.pallas_call(
        _tc_body,
        grid=(2, NB, NRB),
        in_specs=[
            pl.BlockSpec((1, RB, D), lambda l, j, b: (l, b, 0)),
            pl.BlockSpec((W, D), lambda l, j, b: (jnp.where(l == 0, j, 0), 0)),
            pl.BlockSpec((W, D), lambda l, j, b: (jnp.where(l == 1, j, 0), 0)),
        ],
        out_specs=[
            pl.BlockSpec((1, RB, W), lambda l, j, b: (l, b, j)),
            pl.BlockSpec((1, RB, 16), lambda l, j, b: (l, b, 0)),
            pl.BlockSpec((1, RB, 16), lambda l, j, b: (l, b, 0)),
        ],
        out_shape=[
            jax.ShapeDtypeStruct((2, B, N), jnp.float32),
            jax.ShapeDtypeStruct((2, B, 16), jnp.float32),
            jax.ShapeDtypeStruct((2, B, 16), jnp.int32),
        ],
        scratch_shapes=[
            pltpu.VMEM((B, 2 * K), jnp.float32),
            pltpu.VMEM((B, 2 * K), jnp.int32),
            pltpu.VMEM((W, D), jnp.float32),
        ],
    )(feats, mem_0, mem_1)

    wpad = pl.pallas_call(
        _wt_body,
        grid=(2,),
        in_specs=[pl.BlockSpec((1, B, 16), lambda l: (l, 0, 0))],
        out_specs=pl.BlockSpec((1, B, 128), lambda l: (l, 0, 0)),
        out_shape=jax.ShapeDtypeStruct((2, B, 128), jnp.float32),
    )(pvals)
    weights = wpad[:, :, 0]

    idx32 = jnp.pad(pidx[:, :, :K].reshape(NW, VALS_PW), ((0, 0), (0, 16)))
    val32 = jnp.pad(pvals[:, :, :K].reshape(NW, VALS_PW), ((0, 0), (0, 16)))

    soft_flat = pl.kernel(
        _sc_body,
        out_type=jax.ShapeDtypeStruct((2 * B * N,), jnp.float32),
        mesh=plsc.VectorSubcoreMesh(core_axis_name="c", subcore_axis_name="s"),
        compiler_params=pltpu.CompilerParams(needs_layout_passes=False),
        scratch_types=[
            pltpu.VMEM((N,), jnp.float32),
            pltpu.VMEM((VPAD,), jnp.int32),
            pltpu.VMEM((VPAD,), jnp.float32),
        ],
    )(idx32, val32)
    soft = soft_flat.reshape(2, B, N)

    return (soft, sims, weights)


# speedup vs baseline: 1.8995x; 1.4051x over previous
import functools

import jax
import jax.numpy as jnp
from jax import lax
from jax.experimental import pallas as pl
from jax.experimental.pallas import tpu as pltpu
from jax.experimental.pallas import tpu_sc as plsc

B, D, N = 1024, 128, 100000
K = 10
TEMP = 3.0
W = 2048
NB = (N + W - 1) // W
NBPAD = 64
RB = 256
NRB = B // RB
NEG = -3.0e38
IMAX = 2**31 - 1
ZC = float((N - K) * 1e-8 * 18.420680743952367)

NW = 32
ROWS_PW = (2 * B) // NW
VALS_PW = ROWS_PW * K
VPAD = VALS_PW + 16


def _tc_body(feat_ref, mem0_ref, mem1_ref, sims_ref, pv_ref, pi_ref,
             rv, ri, mn, bm, tau):
    l = pl.program_id(0)
    p = pl.program_id(1)
    j = pl.program_id(2)
    b = pl.program_id(3)
    rows = pl.ds(b * RB, RB)

    @pl.when(b == 0)
    def _():
        m = jnp.where(l == 0, mem0_ref[...], mem1_ref[...])
        nrm = jnp.sqrt(jnp.sum(m * m, axis=1, keepdims=True))
        mn[...] = m / jnp.maximum(nrm, 1e-12)

    f = feat_ref[0]
    fn = f / jnp.maximum(
        jnp.sqrt(jnp.sum(f * f, axis=1, keepdims=True)), 1e-12)
    s = lax.dot_general(fn, mn[...], (((1,), (1,)), ((), ())),
                        preferred_element_type=jnp.float32)
    col = j * W + lax.broadcasted_iota(jnp.int32, (RB, W), 1)
    s = jnp.where(col < N, s, NEG)
    bmax = jnp.max(s, axis=1, keepdims=True)

    @pl.when(p == 0)
    def _():
        sims_ref[0, :, :] = s
        lane64 = lax.broadcasted_iota(jnp.int32, (RB, NBPAD), 1)
        old = jnp.where(j == 0, NEG, bm[rows, :])
        bm[rows, :] = jnp.where(lane64 == j, bmax, old)

    @pl.when(p == 1)
    def _():
        @pl.when(j == 0)
        def _():
            rv[rows, :] = jnp.full((RB, 2 * K), NEG, jnp.float32)
            ri[rows, :] = jnp.full((RB, 2 * K), IMAX, jnp.int32)
            lane64 = lax.broadcasted_iota(jnp.int32, (RB, NBPAD), 1)

            def tau_body(t, bv):
                m = jnp.max(bv, axis=1, keepdims=True)
                cand = jnp.where(bv == m, lane64, IMAX)
                am = jnp.min(cand, axis=1, keepdims=True)
                return m, jnp.where(lane64 == am, NEG, bv)

            bv = bm[rows, :]
            m10 = jnp.zeros((RB, 1), jnp.float32)
            for _t in range(K):
                m10, bv = tau_body(_t, bv)
            tau[rows, :] = jnp.broadcast_to(m10, (RB, 8))

        tt = tau[rows, 0:1]
        isc = s >= tt
        cnt = jnp.sum(isc.astype(jnp.int32), axis=1, keepdims=True)
        niter = jnp.minimum(jnp.max(cnt), K)

        @pl.when(niter > 0)
        def _():
            lanes = lax.broadcasted_iota(jnp.int32, (RB, 2 * K), 1)
            sc0 = jnp.where(isc, s, NEG)

            def ext_body(t, carry):
                scv, accv, acci = carry
                m = jnp.max(scv, axis=1, keepdims=True)
                cand = jnp.where(scv == m, col, IMAX)
                am = jnp.min(cand, axis=1, keepdims=True)
                scv = jnp.where(cand == am, NEG, scv)
                accv = jnp.where(lanes == K + t, m, accv)
                acci = jnp.where(lanes == K + t, am, acci)
                return scv, accv, acci

            _, accv, acci = lax.fori_loop(
                0, niter, ext_body, (sc0, rv[rows, :], ri[rows, :]))

            def mrg_body(t, carry):
                a, ai, ov, oi = carry
                m = jnp.max(a, axis=1, keepdims=True)
                cand = jnp.where(a == m, ai, IMAX)
                am = jnp.min(cand, axis=1, keepdims=True)
                a = jnp.where(cand == am, NEG, a)
                ov = jnp.where(lanes == t, m, ov)
                oi = jnp.where(lanes == t, am, oi)
                return a, ai, ov, oi

            ovi = jnp.full((RB, 2 * K), NEG, jnp.float32)
            oii = jnp.full((RB, 2 * K), IMAX, jnp.int32)
            _, _, ov, oi = lax.fori_loop(
                0, K, mrg_body, (accv, acci, ovi, oii))
            rv[rows, :] = ov
            ri[rows, :] = oi

        @pl.when(j == NB - 1)
        def _():
            v16 = rv[rows, 0:16]
            v0 = rv[rows, 0:1]
            e = jnp.exp((v16 - v0) / TEMP)
            z = jnp.sum(e, axis=1, keepdims=True)
            pv_ref[0, :, :] = e / z
            pi_ref[0, :, :] = ri[rows, 0:16]


def _wt_body(pv_ref, w_ref):
    p = pv_ref[0]
    lane = lax.broadcasted_iota(jnp.int32, (B, 16), 1)
    term = jnp.where(lane < K, p * jnp.log(jnp.maximum(p, 1e-30)), 0.0)
    ent = -jnp.sum(term, axis=1, keepdims=True) + ZC
    w = jnp.exp(-ent)
    mw = jnp.sum(w) / B
    w_ref[0, :, :] = jnp.broadcast_to(w / (mw + 1e-8), (B, 128))


def _sc_body(idx_hbm, val_hbm, out_hbm, rowbuf, idxbuf, valbuf):
    wid = lax.axis_index("s") * 2 + lax.axis_index("c")
    pltpu.sync_copy(idx_hbm.at[wid], idxbuf)
    pltpu.sync_copy(val_hbm.at[wid], valbuf)

    zero16 = jnp.zeros((16,), jnp.float32)

    def zbody(i, c):
        rowbuf[pl.ds(i * 16, 16)] = zero16
        return c

    lax.fori_loop(0, N // 16, zbody, 0)

    kmask = lax.iota(jnp.int32, 16) < K

    def rbody(r, c):
        idxv = idxbuf[pl.ds(r * K, 16)]
        valv = valbuf[pl.ds(r * K, 16)]
        plsc.store_scatter(rowbuf, [idxv], valv, mask=kmask)
        base = (wid * ROWS_PW + r) * N
        pltpu.sync_copy(rowbuf, out_hbm.at[pl.ds(base, N)])
        plsc.store_scatter(rowbuf, [idxv], zero16, mask=kmask)
        return c

    lax.fori_loop(0, ROWS_PW, rbody, 0)


def kernel(feat_0, feat_1, mem_0, mem_1):
    feats = jnp.stack([feat_0, feat_1])

    sims, pvals, pidx = pl.pallas_call(
        _tc_body,
        grid=(2, 2, NB, NRB),
        in_specs=[
            pl.BlockSpec((1, RB, D), lambda l, p, j, b: (l, b, 0)),
            pl.BlockSpec(
                (W, D), lambda l, p, j, b: (jnp.where(l == 0, j, 0), 0)),
            pl.BlockSpec(
                (W, D), lambda l, p, j, b: (jnp.where(l == 1, j, 0), 0)),
        ],
        out_specs=[
            pl.BlockSpec(
                (1, RB, W),
                lambda l, p, j, b: (l, jnp.where(p == 0, b, NRB - 1),
                                    jnp.where(p == 0, j, NB - 1))),
            pl.BlockSpec((1, RB, 16), lambda l, p, j, b: (l, b, 0)),
            pl.BlockSpec((1, RB, 16), lambda l, p, j, b: (l, b, 0)),
        ],
        out_shape=[
            jax.ShapeDtypeStruct((2, B, N), jnp.float32),
            jax.ShapeDtypeStruct((2, B, 16), jnp.float32),
            jax.ShapeDtypeStruct((2, B, 16), jnp.int32),
        ],
        scratch_shapes=[
            pltpu.VMEM((B, 2 * K), jnp.float32),
            pltpu.VMEM((B, 2 * K), jnp.int32),
            pltpu.VMEM((W, D), jnp.float32),
            pltpu.VMEM((B, NBPAD), jnp.float32),
            pltpu.VMEM((B, 8), jnp.float32),
        ],
    )(feats, mem_0, mem_1)

    wpad = pl.pallas_call(
        _wt_body,
        grid=(2,),
        in_specs=[pl.BlockSpec((1, B, 16), lambda l: (l, 0, 0))],
        out_specs=pl.BlockSpec((1, B, 128), lambda l: (l, 0, 0)),
        out_shape=jax.ShapeDtypeStruct((2, B, 128), jnp.float32),
    )(pvals)
    weights = wpad[:, :, 0]

    idx32 = jnp.pad(pidx[:, :, :K].reshape(NW, VALS_PW), ((0, 0), (0, 16)))
    val32 = jnp.pad(pvals[:, :, :K].reshape(NW, VALS_PW), ((0, 0), (0, 16)))

    soft_flat = pl.kernel(
        _sc_body,
        out_type=jax.ShapeDtypeStruct((2 * B * N,), jnp.float32),
        mesh=plsc.VectorSubcoreMesh(core_axis_name="c", subcore_axis_name="s"),
        compiler_params=pltpu.CompilerParams(needs_layout_passes=False),
        scratch_types=[
            pltpu.VMEM((N,), jnp.float32),
            pltpu.VMEM((VPAD,), jnp.int32),
            pltpu.VMEM((VPAD,), jnp.float32),
        ],
    )(idx32, val32)
    soft = soft_flat.reshape(2, B, N)

    return (soft, sims, weights)


# speedup vs baseline: 2.0475x; 1.0780x over previous
import functools

import jax
import jax.numpy as jnp
from jax import lax
from jax.experimental import pallas as pl
from jax.experimental.pallas import tpu as pltpu
from jax.experimental.pallas import tpu_sc as plsc

B, D, N = 1024, 128, 100000
K = 10
TEMP = 3.0
W = 4096
NB = (N + W - 1) // W
NBPAD = 32
RB = 256
NRB = B // RB
NEG = -3.0e38
IMAX = 2**31 - 1
ZC = float((N - K) * 1e-8 * 18.420680743952367)

NW = 32
ROWS_PW = (2 * B) // NW
VALS_PW = ROWS_PW * K
VPAD = VALS_PW + 16


def _tc_body(feat_ref, mem0_ref, mem1_ref, sims_ref, pv_ref, pi_ref,
             rv, ri, mn, bm, tau):
    l = pl.program_id(0)
    p = pl.program_id(1)
    j = pl.program_id(2)
    b = pl.program_id(3)
    rows = pl.ds(b * RB, RB)

    @pl.when(b == 0)
    def _():
        m = jnp.where(l == 0, mem0_ref[...], mem1_ref[...])
        nrm = jnp.sqrt(jnp.sum(m * m, axis=1, keepdims=True))
        mn[...] = m / jnp.maximum(nrm, 1e-12)

    f = feat_ref[0]
    fn = f / jnp.maximum(
        jnp.sqrt(jnp.sum(f * f, axis=1, keepdims=True)), 1e-12)
    s = lax.dot_general(fn, mn[...], (((1,), (1,)), ((), ())),
                        preferred_element_type=jnp.float32)
    col = j * W + lax.broadcasted_iota(jnp.int32, (RB, W), 1)
    s = jnp.where(col < N, s, NEG)
    bmax = jnp.max(s, axis=1, keepdims=True)

    @pl.when(p == 0)
    def _():
        sims_ref[0, :, :] = s
        lane64 = lax.broadcasted_iota(jnp.int32, (RB, NBPAD), 1)
        old = jnp.where(j == 0, NEG, bm[rows, :])
        bm[rows, :] = jnp.where(lane64 == j, bmax, old)

    @pl.when(p == 1)
    def _():
        @pl.when(j == 0)
        def _():
            rv[rows, :] = jnp.full((RB, 2 * K), NEG, jnp.float32)
            ri[rows, :] = jnp.full((RB, 2 * K), IMAX, jnp.int32)
            lane64 = lax.broadcasted_iota(jnp.int32, (RB, NBPAD), 1)

            def tau_body(t, bv):
                m = jnp.max(bv, axis=1, keepdims=True)
                cand = jnp.where(bv == m, lane64, IMAX)
                am = jnp.min(cand, axis=1, keepdims=True)
                return m, jnp.where(lane64 == am, NEG, bv)

            bv = bm[rows, :]
            m10 = jnp.zeros((RB, 1), jnp.float32)
            for _t in range(K):
                m10, bv = tau_body(_t, bv)
            tau[rows, :] = jnp.broadcast_to(m10, (RB, 8))

        tt = tau[rows, 0:1]
        isc = s >= tt
        cnt = jnp.sum(isc.astype(jnp.int32), axis=1, keepdims=True)
        niter = jnp.minimum(jnp.max(cnt), K)

        @pl.when(niter > 0)
        def _():
            lanes = lax.broadcasted_iota(jnp.int32, (RB, 2 * K), 1)
            sc0 = jnp.where(isc, s, NEG)

            def ext_body(t, carry):
                scv, accv, acci = carry
                m = jnp.max(scv, axis=1, keepdims=True)
                cand = jnp.where(scv == m, col, IMAX)
                am = jnp.min(cand, axis=1, keepdims=True)
                scv = jnp.where(cand == am, NEG, scv)
                accv = jnp.where(lanes == K + t, m, accv)
                acci = jnp.where(lanes == K + t, am, acci)
                return scv, accv, acci

            _, accv, acci = lax.fori_loop(
                0, niter, ext_body, (sc0, rv[rows, :], ri[rows, :]))

            def mrg_body(t, carry):
                a, ai, ov, oi = carry
                m = jnp.max(a, axis=1, keepdims=True)
                cand = jnp.where(a == m, ai, IMAX)
                am = jnp.min(cand, axis=1, keepdims=True)
                a = jnp.where(cand == am, NEG, a)
                ov = jnp.where(lanes == t, m, ov)
                oi = jnp.where(lanes == t, am, oi)
                return a, ai, ov, oi

            ovi = jnp.full((RB, 2 * K), NEG, jnp.float32)
            oii = jnp.full((RB, 2 * K), IMAX, jnp.int32)
            _, _, ov, oi = lax.fori_loop(
                0, K, mrg_body, (accv, acci, ovi, oii))
            rv[rows, :] = ov
            ri[rows, :] = oi

        @pl.when(j == NB - 1)
        def _():
            v16 = rv[rows, 0:16]
            v0 = rv[rows, 0:1]
            e = jnp.exp((v16 - v0) / TEMP)
            z = jnp.sum(e, axis=1, keepdims=True)
            pv_ref[0, :, :] = e / z
            pi_ref[0, :, :] = ri[rows, 0:16]


def _wt_body(pv_ref, w_ref):
    p = pv_ref[0]
    lane = lax.broadcasted_iota(jnp.int32, (B, 16), 1)
    term = jnp.where(lane < K, p * jnp.log(jnp.maximum(p, 1e-30)), 0.0)
    ent = -jnp.sum(term, axis=1, keepdims=True) + ZC
    w = jnp.exp(-ent)
    mw = jnp.sum(w) / B
    w_ref[0, :, :] = jnp.broadcast_to(w / (mw + 1e-8), (B, 128))


def _sc_body(idx_hbm, val_hbm, out_hbm, rowbuf, idxbuf, valbuf):
    wid = lax.axis_index("s") * 2 + lax.axis_index("c")
    pltpu.sync_copy(idx_hbm.at[wid], idxbuf)
    pltpu.sync_copy(val_hbm.at[wid], valbuf)

    zero16 = jnp.zeros((16,), jnp.float32)

    def zbody(i, c):
        rowbuf[pl.ds(i * 16, 16)] = zero16
        return c

    lax.fori_loop(0, N // 16, zbody, 0)

    kmask = lax.iota(jnp.int32, 16) < K

    def rbody(r, c):
        idxv = idxbuf[pl.ds(r * K, 16)]
        valv = valbuf[pl.ds(r * K, 16)]
        plsc.store_scatter(rowbuf, [idxv], valv, mask=kmask)
        base = (wid * ROWS_PW + r) * N
        pltpu.sync_copy(rowbuf, out_hbm.at[pl.ds(base, N)])
        plsc.store_scatter(rowbuf, [idxv], zero16, mask=kmask)
        return c

    lax.fori_loop(0, ROWS_PW, rbody, 0)


def kernel(feat_0, feat_1, mem_0, mem_1):
    feats = jnp.stack([feat_0, feat_1])

    sims, pvals, pidx = pl.pallas_call(
        _tc_body,
        grid=(2, 2, NB, NRB),
        in_specs=[
            pl.BlockSpec((1, RB, D), lambda l, p, j, b: (l, b, 0)),
            pl.BlockSpec(
                (W, D), lambda l, p, j, b: (jnp.where(l == 0, j, 0), 0)),
            pl.BlockSpec(
                (W, D), lambda l, p, j, b: (jnp.where(l == 1, j, 0), 0)),
        ],
        out_specs=[
            pl.BlockSpec(
                (1, RB, W),
                lambda l, p, j, b: (l, jnp.where(p == 0, b, NRB - 1),
                                    jnp.where(p == 0, j, NB - 1))),
            pl.BlockSpec((1, RB, 16), lambda l, p, j, b: (l, b, 0)),
            pl.BlockSpec((1, RB, 16), lambda l, p, j, b: (l, b, 0)),
        ],
        out_shape=[
            jax.ShapeDtypeStruct((2, B, N), jnp.float32),
            jax.ShapeDtypeStruct((2, B, 16), jnp.float32),
            jax.ShapeDtypeStruct((2, B, 16), jnp.int32),
        ],
        scratch_shapes=[
            pltpu.VMEM((B, 2 * K), jnp.float32),
            pltpu.VMEM((B, 2 * K), jnp.int32),
            pltpu.VMEM((W, D), jnp.float32),
            pltpu.VMEM((B, NBPAD), jnp.float32),
            pltpu.VMEM((B, 8), jnp.float32),
        ],
    )(feats, mem_0, mem_1)

    wpad = pl.pallas_call(
        _wt_body,
        grid=(2,),
        in_specs=[pl.BlockSpec((1, B, 16), lambda l: (l, 0, 0))],
        out_specs=pl.BlockSpec((1, B, 128), lambda l: (l, 0, 0)),
        out_shape=jax.ShapeDtypeStruct((2, B, 128), jnp.float32),
    )(pvals)
    weights = wpad[:, :, 0]

    idx32 = jnp.pad(pidx[:, :, :K].reshape(NW, VALS_PW), ((0, 0), (0, 16)))
    val32 = jnp.pad(pvals[:, :, :K].reshape(NW, VALS_PW), ((0, 0), (0, 16)))

    soft_flat = pl.kernel(
        _sc_body,
        out_type=jax.ShapeDtypeStruct((2 * B * N,), jnp.float32),
        mesh=plsc.VectorSubcoreMesh(core_axis_name="c", subcore_axis_name="s"),
        compiler_params=pltpu.CompilerParams(needs_layout_passes=False),
        scratch_types=[
            pltpu.VMEM((N,), jnp.float32),
            pltpu.VMEM((VPAD,), jnp.int32),
            pltpu.VMEM((VPAD,), jnp.float32),
        ],
    )(idx32, val32)
    soft = soft_flat.reshape(2, B, N)

    return (soft, sims, weights)


# speedup vs baseline: 2.5793x; 1.2597x over previous
import functools

import jax
import jax.numpy as jnp
from jax import lax
from jax.experimental import pallas as pl
from jax.experimental.pallas import tpu as pltpu
from jax.experimental.pallas import tpu_sc as plsc

B, D, N = 1024, 128, 100000
K = 10
TEMP = 3.0
W = 4096
NB = (N + W - 1) // W
NBPAD = 32
RB = 256
NRB = B // RB
NEG = -3.0e38
IMAX = 2**31 - 1
ZC = float((N - K) * 1e-8 * 18.420680743952367)

NW = 32
ROWS_PW = (2 * B) // NW
VALS_PW = ROWS_PW * K
VPAD = VALS_PW + 16


def _tc_body(feat_ref, mem0_ref, mem1_ref, sims_ref, pv_ref, pi_ref,
             rv, ri, mn, bm, tau):
    l = pl.program_id(0)
    p = pl.program_id(1)
    j = pl.program_id(2)
    b = pl.program_id(3)
    rows = pl.ds(b * RB, RB)

    @pl.when(b == 0)
    def _():
        m = jnp.where(l == 0, mem0_ref[...], mem1_ref[...])
        nrm = jnp.sqrt(jnp.sum(m * m, axis=1, keepdims=True))
        mn[...] = m / jnp.maximum(nrm, 1e-12)

    f = feat_ref[0]
    fn = f / jnp.maximum(
        jnp.sqrt(jnp.sum(f * f, axis=1, keepdims=True)), 1e-12)
    s = lax.dot_general(fn, mn[...], (((1,), (1,)), ((), ())),
                        preferred_element_type=jnp.float32)
    col = j * W + lax.broadcasted_iota(jnp.int32, (RB, W), 1)
    s = jnp.where(col < N, s, NEG)
    bmax = jnp.max(s, axis=1, keepdims=True)

    @pl.when(p == 0)
    def _():
        sims_ref[0, :, :] = s
        lane64 = lax.broadcasted_iota(jnp.int32, (RB, NBPAD), 1)
        old = jnp.where(j == 0, NEG, bm[rows, :])
        bm[rows, :] = jnp.where(lane64 == j, bmax, old)

    @pl.when(p == 1)
    def _():
        @pl.when(j == 0)
        def _():
            rv[rows, :] = jnp.full((RB, 2 * K), NEG, jnp.float32)
            ri[rows, :] = jnp.full((RB, 2 * K), IMAX, jnp.int32)
            lane64 = lax.broadcasted_iota(jnp.int32, (RB, NBPAD), 1)

            def tau_body(t, bv):
                m = jnp.max(bv, axis=1, keepdims=True)
                cand = jnp.where(bv == m, lane64, IMAX)
                am = jnp.min(cand, axis=1, keepdims=True)
                return m, jnp.where(lane64 == am, NEG, bv)

            bv = bm[rows, :]
            m10 = jnp.zeros((RB, 1), jnp.float32)
            for _t in range(K):
                m10, bv = tau_body(_t, bv)
            tau[rows, :] = jnp.broadcast_to(m10, (RB, 8))

        tt = tau[rows, 0:1]
        isc = s >= tt
        cnt = jnp.sum(isc.astype(jnp.int32), axis=1, keepdims=True)
        niter = jnp.minimum(jnp.max(cnt), K)

        @pl.when(niter > 0)
        def _():
            lanes = lax.broadcasted_iota(jnp.int32, (RB, 2 * K), 1)
            sc0 = jnp.where(isc, s, NEG)

            def ext_body(t, carry):
                scv, accv, acci = carry
                m = jnp.max(scv, axis=1, keepdims=True)
                cand = jnp.where(scv == m, col, IMAX)
                am = jnp.min(cand, axis=1, keepdims=True)
                scv = jnp.where(cand == am, NEG, scv)
                accv = jnp.where(lanes == K + t, m, accv)
                acci = jnp.where(lanes == K + t, am, acci)
                return scv, accv, acci

            _, accv, acci = lax.fori_loop(
                0, niter, ext_body, (sc0, rv[rows, :], ri[rows, :]))

            def mrg_body(t, carry):
                a, ai, ov, oi = carry
                m = jnp.max(a, axis=1, keepdims=True)
                cand = jnp.where(a == m, ai, IMAX)
                am = jnp.min(cand, axis=1, keepdims=True)
                a = jnp.where(cand == am, NEG, a)
                ov = jnp.where(lanes == t, m, ov)
                oi = jnp.where(lanes == t, am, oi)
                return a, ai, ov, oi

            ovi = jnp.full((RB, 2 * K), NEG, jnp.float32)
            oii = jnp.full((RB, 2 * K), IMAX, jnp.int32)
            _, _, ov, oi = lax.fori_loop(
                0, K, mrg_body, (accv, acci, ovi, oii))
            rv[rows, :] = ov
            ri[rows, :] = oi

        @pl.when(j == NB - 1)
        def _():
            v16 = rv[rows, 0:16]
            v0 = rv[rows, 0:1]
            e = jnp.exp((v16 - v0) / TEMP)
            z = jnp.sum(e, axis=1, keepdims=True)
            pv_ref[0, :, :] = e / z
            pi_ref[0, :, :] = ri[rows, 0:16]


def _wt_body(pv_ref, w_ref):
    p = pv_ref[0]
    lane = lax.broadcasted_iota(jnp.int32, (B, 16), 1)
    term = jnp.where(lane < K, p * jnp.log(jnp.maximum(p, 1e-30)), 0.0)
    ent = -jnp.sum(term, axis=1, keepdims=True) + ZC
    w = jnp.exp(-ent)
    mw = jnp.sum(w) / B
    w_ref[0, :, :] = jnp.broadcast_to(w / (mw + 1e-8), (B, 128))


def _sc_body(idx_hbm, val_hbm, out_hbm, rowbuf, idxbuf, valbuf):
    wid = lax.axis_index("s") * 2 + lax.axis_index("c")
    pltpu.sync_copy(idx_hbm.at[wid], idxbuf)
    pltpu.sync_copy(val_hbm.at[wid], valbuf)

    zero16 = jnp.zeros((16,), jnp.float32)

    def zbody(i, c):
        rowbuf[pl.ds(i * 16, 16)] = zero16
        return c

    lax.fori_loop(0, N // 16, zbody, 0)

    kmask = lax.iota(jnp.int32, 16) < K

    def rbody(r, c):
        idxv = idxbuf[pl.ds(r * K, 16)]
        valv = valbuf[pl.ds(r * K, 16)]
        plsc.store_scatter(rowbuf, [idxv], valv, mask=kmask)
        pltpu.sync_copy(rowbuf, out_hbm.at[wid * ROWS_PW + r])
        plsc.store_scatter(rowbuf, [idxv], zero16, mask=kmask)
        return c

    lax.fori_loop(0, ROWS_PW, rbody, 0)


def kernel(feat_0, feat_1, mem_0, mem_1):
    feats = jnp.stack([feat_0, feat_1])

    sims, pvals, pidx = pl.pallas_call(
        _tc_body,
        grid=(2, 2, NB, NRB),
        in_specs=[
            pl.BlockSpec((1, RB, D), lambda l, p, j, b: (l, b, 0)),
            pl.BlockSpec(
                (W, D), lambda l, p, j, b: (jnp.where(l == 0, j, 0), 0)),
            pl.BlockSpec(
                (W, D), lambda l, p, j, b: (jnp.where(l == 1, j, 0), 0)),
        ],
        out_specs=[
            pl.BlockSpec(
                (1, RB, W),
                lambda l, p, j, b: (l, jnp.where(p == 0, b, NRB - 1),
                                    jnp.where(p == 0, j, NB - 1))),
            pl.BlockSpec((1, RB, 16), lambda l, p, j, b: (l, b, 0)),
            pl.BlockSpec((1, RB, 16), lambda l, p, j, b: (l, b, 0)),
        ],
        out_shape=[
            jax.ShapeDtypeStruct((2, B, N), jnp.float32),
            jax.ShapeDtypeStruct((2, B, 16), jnp.float32),
            jax.ShapeDtypeStruct((2, B, 16), jnp.int32),
        ],
        scratch_shapes=[
            pltpu.VMEM((B, 2 * K), jnp.float32),
            pltpu.VMEM((B, 2 * K), jnp.int32),
            pltpu.VMEM((W, D), jnp.float32),
            pltpu.VMEM((B, NBPAD), jnp.float32),
            pltpu.VMEM((B, 8), jnp.float32),
        ],
    )(feats, mem_0, mem_1)

    wpad = pl.pallas_call(
        _wt_body,
        grid=(2,),
        in_specs=[pl.BlockSpec((1, B, 16), lambda l: (l, 0, 0))],
        out_specs=pl.BlockSpec((1, B, 128), lambda l: (l, 0, 0)),
        out_shape=jax.ShapeDtypeStruct((2, B, 128), jnp.float32),
    )(pvals)
    weights = wpad[:, :, 0]

    idx32 = jnp.pad(pidx[:, :, :K].reshape(NW, VALS_PW), ((0, 0), (0, 16)))
    val32 = jnp.pad(pvals[:, :, :K].reshape(NW, VALS_PW), ((0, 0), (0, 16)))

    soft2d = pl.kernel(
        _sc_body,
        out_type=jax.ShapeDtypeStruct((2 * B, N), jnp.float32),
        mesh=plsc.VectorSubcoreMesh(core_axis_name="c", subcore_axis_name="s"),
        compiler_params=pltpu.CompilerParams(
            needs_layout_passes=False, use_tc_tiling_on_sc=True),
        scratch_types=[
            pltpu.VMEM((N,), jnp.float32),
            pltpu.VMEM((VPAD,), jnp.int32),
            pltpu.VMEM((VPAD,), jnp.float32),
        ],
    )(idx32, val32)
    soft = soft2d.reshape(2, B, N)

    return (soft, sims, weights)


# speedup vs baseline: 2.9434x; 1.1412x over previous
import functools

import jax
import jax.numpy as jnp
from jax import lax
from jax.experimental import pallas as pl
from jax.experimental.pallas import tpu as pltpu
from jax.experimental.pallas import tpu_sc as plsc

B, D, N = 1024, 128, 100000
K = 10
TEMP = 3.0
W = 4096
NB = (N + W - 1) // W
NBPAD = 32
RB = 256
NRB = B // RB
NEG = -3.0e38
IMAX = 2**31 - 1
ZC = float((N - K) * 1e-8 * 18.420680743952367)

NW = 32
ROWS_PW = (2 * B) // NW
VALS_PW = ROWS_PW * K
VPAD = VALS_PW + 16


def _tc_body(feat_ref, mem0_ref, mem1_ref, sims_ref, pv_ref, pi_ref,
             rv, ri, mn, bm, tau):
    l = pl.program_id(0)
    p = pl.program_id(1)
    j = pl.program_id(2)
    b = pl.program_id(3)
    rows = pl.ds(b * RB, RB)

    @pl.when(b == 0)
    def _():
        m = jnp.where(l == 0, mem0_ref[...], mem1_ref[...])
        nrm = jnp.sqrt(jnp.sum(m * m, axis=1, keepdims=True))
        mn[...] = m / jnp.maximum(nrm, 1e-12)

    f = feat_ref[0]
    fn = f / jnp.maximum(
        jnp.sqrt(jnp.sum(f * f, axis=1, keepdims=True)), 1e-12)
    s = lax.dot_general(fn, mn[...], (((1,), (1,)), ((), ())),
                        preferred_element_type=jnp.float32)
    col = j * W + lax.broadcasted_iota(jnp.int32, (RB, W), 1)
    s = jnp.where(col < N, s, NEG)
    bmax = jnp.max(s, axis=1, keepdims=True)

    @pl.when(p == 0)
    def _():
        sims_ref[0, :, :] = s
        lane64 = lax.broadcasted_iota(jnp.int32, (RB, NBPAD), 1)
        old = jnp.where(j == 0, NEG, bm[rows, :])
        bm[rows, :] = jnp.where(lane64 == j, bmax, old)

    @pl.when(p == 1)
    def _():
        @pl.when(j == 0)
        def _():
            rv[rows, :] = jnp.full((RB, 16), NEG, jnp.float32)
            ri[rows, :] = jnp.full((RB, 16), IMAX, jnp.int32)
            lane64 = lax.broadcasted_iota(jnp.int32, (RB, NBPAD), 1)

            def tau_body(t, bv):
                m = jnp.max(bv, axis=1, keepdims=True)
                cand = jnp.where(bv == m, lane64, IMAX)
                am = jnp.min(cand, axis=1, keepdims=True)
                return m, jnp.where(lane64 == am, NEG, bv)

            bv = bm[rows, :]
            m10 = jnp.zeros((RB, 1), jnp.float32)
            for _t in range(K):
                m10, bv = tau_body(_t, bv)
            tau[rows, :] = jnp.broadcast_to(m10, (RB, 8))

        tt = tau[rows, 0:1]
        isc = s >= tt
        cnt = jnp.sum(isc.astype(jnp.int32), axis=1, keepdims=True)
        niter = jnp.minimum(jnp.max(cnt), K)

        @pl.when(niter > 0)
        def _():
            sc0 = jnp.where(isc, s, NEG)
            ones1 = jnp.ones((RB, 1), jnp.int32)

            def ext_body(t, carry):
                scv, lv, li = carry
                m = jnp.max(scv, axis=1, keepdims=True)
                cand = jnp.where(scv == m, col, IMAX)
                am = jnp.min(cand, axis=1, keepdims=True)
                scv = jnp.where(cand == am, NEG, scv)
                ge = (lv > m) | ((lv == m) & (li < am))
                gei = jnp.where(ge, 1, 0)
                ges = jnp.concatenate([ones1, gei[:, :15]], axis=1) > 0
                lvp = jnp.concatenate([lv[:, 0:1], lv[:, :15]], axis=1)
                lip = jnp.concatenate([li[:, 0:1], li[:, :15]], axis=1)
                lv = jnp.where(ge, lv, jnp.where(ges, m, lvp))
                li = jnp.where(ge, li, jnp.where(ges, am, lip))
                return scv, lv, li

            _, lv, li = lax.fori_loop(
                0, niter, ext_body, (sc0, rv[rows, :], ri[rows, :]))
            rv[rows, :] = lv
            ri[rows, :] = li

        @pl.when(j == NB - 1)
        def _():
            lane16 = lax.broadcasted_iota(jnp.int32, (RB, 16), 1)
            v16 = rv[rows, :]
            v0 = rv[rows, 0:1]
            e = jnp.where(lane16 < K, jnp.exp((v16 - v0) / TEMP), 0.0)
            z = jnp.sum(e, axis=1, keepdims=True)
            pv_ref[0, :, :] = e / z
            pi_ref[0, :, :] = ri[rows, :]


def _wt_body(pv_ref, w_ref):
    p = pv_ref[0]
    lane = lax.broadcasted_iota(jnp.int32, (B, 16), 1)
    term = jnp.where(lane < K, p * jnp.log(jnp.maximum(p, 1e-30)), 0.0)
    ent = -jnp.sum(term, axis=1, keepdims=True) + ZC
    w = jnp.exp(-ent)
    mw = jnp.sum(w) / B
    w_ref[0, :, :] = jnp.broadcast_to(w / (mw + 1e-8), (B, 128))


def _sc_body(idx_hbm, val_hbm, out_hbm, rowbuf, idxbuf, valbuf):
    wid = lax.axis_index("s") * 2 + lax.axis_index("c")
    pltpu.sync_copy(idx_hbm.at[wid], idxbuf)
    pltpu.sync_copy(val_hbm.at[wid], valbuf)

    zero16 = jnp.zeros((16,), jnp.float32)

    def zbody(i, c):
        rowbuf[pl.ds(i * 16, 16)] = zero16
        return c

    lax.fori_loop(0, N // 16, zbody, 0)

    kmask = lax.iota(jnp.int32, 16) < K

    def rbody(r, c):
        idxv = idxbuf[pl.ds(r * K, 16)]
        valv = valbuf[pl.ds(r * K, 16)]
        plsc.store_scatter(rowbuf, [idxv], valv, mask=kmask)
        pltpu.sync_copy(rowbuf, out_hbm.at[wid * ROWS_PW + r])
        plsc.store_scatter(rowbuf, [idxv], zero16, mask=kmask)
        return c

    lax.fori_loop(0, ROWS_PW, rbody, 0)


def kernel(feat_0, feat_1, mem_0, mem_1):
    feats = jnp.stack([feat_0, feat_1])

    sims, pvals, pidx = pl.pallas_call(
        _tc_body,
        grid=(2, 2, NB, NRB),
        in_specs=[
            pl.BlockSpec((1, RB, D), lambda l, p, j, b: (l, b, 0)),
            pl.BlockSpec(
                (W, D), lambda l, p, j, b: (jnp.where(l == 0, j, 0), 0)),
            pl.BlockSpec(
                (W, D), lambda l, p, j, b: (jnp.where(l == 1, j, 0), 0)),
        ],
        out_specs=[
            pl.BlockSpec(
                (1, RB, W),
                lambda l, p, j, b: (l, jnp.where(p == 0, b, NRB - 1),
                                    jnp.where(p == 0, j, NB - 1))),
            pl.BlockSpec((1, RB, 16), lambda l, p, j, b: (l, b, 0)),
            pl.BlockSpec((1, RB, 16), lambda l, p, j, b: (l, b, 0)),
        ],
        out_shape=[
            jax.ShapeDtypeStruct((2, B, N), jnp.float32),
            jax.ShapeDtypeStruct((2, B, 16), jnp.float32),
            jax.ShapeDtypeStruct((2, B, 16), jnp.int32),
        ],
        scratch_shapes=[
            pltpu.VMEM((B, 16), jnp.float32),
            pltpu.VMEM((B, 16), jnp.int32),
            pltpu.VMEM((W, D), jnp.float32),
            pltpu.VMEM((B, NBPAD), jnp.float32),
            pltpu.VMEM((B, 8), jnp.float32),
        ],
    )(feats, mem_0, mem_1)

    wpad = pl.pallas_call(
        _wt_body,
        grid=(2,),
        in_specs=[pl.BlockSpec((1, B, 16), lambda l: (l, 0, 0))],
        out_specs=pl.BlockSpec((1, B, 128), lambda l: (l, 0, 0)),
        out_shape=jax.ShapeDtypeStruct((2, B, 128), jnp.float32),
    )(pvals)
    weights = wpad[:, :, 0]

    idx32 = jnp.pad(pidx[:, :, :K].reshape(NW, VALS_PW), ((0, 0), (0, 16)))
    val32 = jnp.pad(pvals[:, :, :K].reshape(NW, VALS_PW), ((0, 0), (0, 16)))

    soft2d = pl.kernel(
        _sc_body,
        out_type=jax.ShapeDtypeStruct((2 * B, N), jnp.float32),
        mesh=plsc.VectorSubcoreMesh(core_axis_name="c", subcore_axis_name="s"),
        compiler_params=pltpu.CompilerParams(
            needs_layout_passes=False, use_tc_tiling_on_sc=True),
        scratch_types=[
            pltpu.VMEM((N,), jnp.float32),
            pltpu.VMEM((VPAD,), jnp.int32),
            pltpu.VMEM((VPAD,), jnp.float32),
        ],
    )(idx32, val32)
    soft = soft2d.reshape(2, B, N)

    return (soft, sims, weights)


# speedup vs baseline: 3.6810x; 1.2506x over previous
import functools

import jax
import jax.numpy as jnp
from jax import lax
from jax.experimental import pallas as pl
from jax.experimental.pallas import tpu as pltpu
from jax.experimental.pallas import tpu_sc as plsc

B, D, N = 1024, 128, 100000
K = 10
TEMP = 3.0
W = 4096
NB = (N + W - 1) // W
NBPAD = 32
RB = 256
NRB = B // RB
NEG = -3.0e38
IMAX = 2**31 - 1
ZC = float((N - K) * 1e-8 * 18.420680743952367)

NW = 32
ROWS_PW = (2 * B) // NW
VALS_PW = ROWS_PW * K
VPAD = VALS_PW + 16


def _tc_body(feat_ref, mem0_ref, mem1_ref, sims_ref, pv_ref, pi_ref,
             rv, ri, mn, bm, tau):
    l = pl.program_id(0)
    p = pl.program_id(1)
    j = pl.program_id(2)
    b = pl.program_id(3)
    rows = pl.ds(b * RB, RB)

    @pl.when(b == 0)
    def _():
        m = jnp.where(l == 0, mem0_ref[...], mem1_ref[...])
        nrm = jnp.sqrt(jnp.sum(m * m, axis=1, keepdims=True))
        mn[...] = m / jnp.maximum(nrm, 1e-12)

    f = feat_ref[0]
    fn = f / jnp.maximum(
        jnp.sqrt(jnp.sum(f * f, axis=1, keepdims=True)), 1e-12)
    s = lax.dot_general(fn, mn[...], (((1,), (1,)), ((), ())),
                        preferred_element_type=jnp.float32)
    col = j * W + lax.broadcasted_iota(jnp.int32, (RB, W), 1)
    s = jnp.where(col < N, s, NEG)
    bmax = jnp.max(s, axis=1, keepdims=True)

    @pl.when(p == 0)
    def _():
        sims_ref[0, :, :] = s
        lane64 = lax.broadcasted_iota(jnp.int32, (RB, NBPAD), 1)
        old = jnp.where(j == 0, NEG, bm[rows, :])
        bm[rows, :] = jnp.where(lane64 == j, bmax, old)

    @pl.when(p == 1)
    def _():
        @pl.when(j == 0)
        def _():
            rv[rows, :] = jnp.full((RB, 16), NEG, jnp.float32)
            ri[rows, :] = jnp.full((RB, 16), IMAX, jnp.int32)
            lane64 = lax.broadcasted_iota(jnp.int32, (RB, NBPAD), 1)

            def tau_body(t, bv):
                m = jnp.max(bv, axis=1, keepdims=True)
                cand = jnp.where(bv == m, lane64, IMAX)
                am = jnp.min(cand, axis=1, keepdims=True)
                return m, jnp.where(lane64 == am, NEG, bv)

            bv = bm[rows, :]
            m10 = jnp.zeros((RB, 1), jnp.float32)
            for _t in range(K):
                m10, bv = tau_body(_t, bv)
            tau[rows, :] = jnp.broadcast_to(m10, (RB, 8))

        tt = tau[rows, 0:1]
        ones1 = jnp.ones((RB, 1), jnp.int32)
        lane128 = lax.broadcasted_iota(jnp.int32, (RB, 128), 1)

        def _insert(lv, li, m, am):
            ge = (lv > m) | ((lv == m) & (li < am))
            gei = jnp.where(ge, 1, 0)
            ges = jnp.concatenate([ones1, gei[:, :15]], axis=1) > 0
            lvp = jnp.concatenate([lv[:, 0:1], lv[:, :15]], axis=1)
            lip = jnp.concatenate([li[:, 0:1], li[:, :15]], axis=1)
            lv = jnp.where(ge, lv, jnp.where(ges, m, lvp))
            li = jnp.where(ge, li, jnp.where(ges, am, lip))
            return lv, li

        f1 = jnp.full((RB, 128), NEG, jnp.float32)
        f2 = jnp.full((RB, 128), NEG, jnp.float32)
        g1 = jnp.zeros((RB, 128), jnp.int32)
        g2 = jnp.zeros((RB, 128), jnp.int32)
        cl = jnp.zeros((RB, 128), jnp.int32)
        for g in range(W // 128):
            t = s[:, g * 128:(g + 1) * 128]
            c1 = t > f1
            c2 = t > f2
            f2 = jnp.where(c1, f1, jnp.where(c2, t, f2))
            g2 = jnp.where(c1, g1, jnp.where(c2, g, g2))
            f1 = jnp.where(c1, t, f1)
            g1 = jnp.where(c1, g, g1)
            cl = cl + jnp.where(t >= tt, 1, 0)
        cnt = jnp.sum(cl, axis=1, keepdims=True)
        niter = jnp.minimum(jnp.max(cnt), K)
        bad = jnp.max(cl) >= 3

        @pl.when((niter > 0) & jnp.logical_not(bad))
        def _():
            def fast_body(it, carry):
                q1, h1, q2, h2, lv, li = carry
                m = jnp.max(q1, axis=1, keepdims=True)
                candc = jnp.where(q1 == m, h1 * 128 + lane128, IMAX)
                amc = jnp.min(candc, axis=1, keepdims=True)
                al = jnp.bitwise_and(amc, 127)
                hit = lane128 == al
                q1 = jnp.where(hit, q2, q1)
                h1 = jnp.where(hit, h2, h1)
                q2 = jnp.where(hit, NEG, q2)
                h2 = jnp.where(hit, 0, h2)
                lv, li = _insert(lv, li, m, j * W + amc)
                return q1, h1, q2, h2, lv, li

            out = lax.fori_loop(
                0, niter, fast_body,
                (f1, g1, f2, g2, rv[rows, :], ri[rows, :]))
            rv[rows, :] = out[4]
            ri[rows, :] = out[5]

        @pl.when((niter > 0) & bad)
        def _():
            sc0 = jnp.where(s >= tt, s, NEG)

            def ext_body(t, carry):
                scv, lv, li = carry
                m = jnp.max(scv, axis=1, keepdims=True)
                cand = jnp.where(scv == m, col, IMAX)
                am = jnp.min(cand, axis=1, keepdims=True)
                scv = jnp.where(cand == am, NEG, scv)
                lv, li = _insert(lv, li, m, am)
                return scv, lv, li

            _, lv, li = lax.fori_loop(
                0, niter, ext_body, (sc0, rv[rows, :], ri[rows, :]))
            rv[rows, :] = lv
            ri[rows, :] = li

        @pl.when(j == NB - 1)
        def _():
            lane16 = lax.broadcasted_iota(jnp.int32, (RB, 16), 1)
            v16 = rv[rows, :]
            v0 = rv[rows, 0:1]
            e = jnp.where(lane16 < K, jnp.exp((v16 - v0) / TEMP), 0.0)
            z = jnp.sum(e, axis=1, keepdims=True)
            pv_ref[0, :, :] = e / z
            pi_ref[0, :, :] = ri[rows, :]


def _wt_body(pv_ref, w_ref):
    p = pv_ref[0]
    lane = lax.broadcasted_iota(jnp.int32, (B, 16), 1)
    term = jnp.where(lane < K, p * jnp.log(jnp.maximum(p, 1e-30)), 0.0)
    ent = -jnp.sum(term, axis=1, keepdims=True) + ZC
    w = jnp.exp(-ent)
    mw = jnp.sum(w) / B
    w_ref[0, :, :] = jnp.broadcast_to(w / (mw + 1e-8), (B, 128))


def _sc_body(idx_hbm, val_hbm, out_hbm, rowbuf, idxbuf, valbuf):
    wid = lax.axis_index("s") * 2 + lax.axis_index("c")
    pltpu.sync_copy(idx_hbm.at[wid], idxbuf)
    pltpu.sync_copy(val_hbm.at[wid], valbuf)

    zero16 = jnp.zeros((16,), jnp.float32)

    def zbody(i, c):
        rowbuf[pl.ds(i * 16, 16)] = zero16
        return c

    lax.fori_loop(0, N // 16, zbody, 0)

    kmask = lax.iota(jnp.int32, 16) < K

    def rbody(r, c):
        idxv = idxbuf[pl.ds(r * K, 16)]
        valv = valbuf[pl.ds(r * K, 16)]
        plsc.store_scatter(rowbuf, [idxv], valv, mask=kmask)
        pltpu.sync_copy(rowbuf, out_hbm.at[wid * ROWS_PW + r])
        plsc.store_scatter(rowbuf, [idxv], zero16, mask=kmask)
        return c

    lax.fori_loop(0, ROWS_PW, rbody, 0)


def kernel(feat_0, feat_1, mem_0, mem_1):
    feats = jnp.stack([feat_0, feat_1])

    sims, pvals, pidx = pl.pallas_call(
        _tc_body,
        grid=(2, 2, NB, NRB),
        in_specs=[
            pl.BlockSpec((1, RB, D), lambda l, p, j, b: (l, b, 0)),
            pl.BlockSpec(
                (W, D), lambda l, p, j, b: (jnp.where(l == 0, j, 0), 0)),
            pl.BlockSpec(
                (W, D), lambda l, p, j, b: (jnp.where(l == 1, j, 0), 0)),
        ],
        out_specs=[
            pl.BlockSpec(
                (1, RB, W),
                lambda l, p, j, b: (l, jnp.where(p == 0, b, NRB - 1),
                                    jnp.where(p == 0, j, NB - 1))),
            pl.BlockSpec((1, RB, 16), lambda l, p, j, b: (l, b, 0)),
            pl.BlockSpec((1, RB, 16), lambda l, p, j, b: (l, b, 0)),
        ],
        out_shape=[
            jax.ShapeDtypeStruct((2, B, N), jnp.float32),
            jax.ShapeDtypeStruct((2, B, 16), jnp.float32),
            jax.ShapeDtypeStruct((2, B, 16), jnp.int32),
        ],
        scratch_shapes=[
            pltpu.VMEM((B, 16), jnp.float32),
            pltpu.VMEM((B, 16), jnp.int32),
            pltpu.VMEM((W, D), jnp.float32),
            pltpu.VMEM((B, NBPAD), jnp.float32),
            pltpu.VMEM((B, 8), jnp.float32),
        ],
    )(feats, mem_0, mem_1)

    wpad = pl.pallas_call(
        _wt_body,
        grid=(2,),
        in_specs=[pl.BlockSpec((1, B, 16), lambda l: (l, 0, 0))],
        out_specs=pl.BlockSpec((1, B, 128), lambda l: (l, 0, 0)),
        out_shape=jax.ShapeDtypeStruct((2, B, 128), jnp.float32),
    )(pvals)
    weights = wpad[:, :, 0]

    idx32 = jnp.pad(pidx[:, :, :K].reshape(NW, VALS_PW), ((0, 0), (0, 16)))
    val32 = jnp.pad(pvals[:, :, :K].reshape(NW, VALS_PW), ((0, 0), (0, 16)))

    soft2d = pl.kernel(
        _sc_body,
        out_type=jax.ShapeDtypeStruct((2 * B, N), jnp.float32),
        mesh=plsc.VectorSubcoreMesh(core_axis_name="c", subcore_axis_name="s"),
        compiler_params=pltpu.CompilerParams(
            needs_layout_passes=False, use_tc_tiling_on_sc=True),
        scratch_types=[
            pltpu.VMEM((N,), jnp.float32),
            pltpu.VMEM((VPAD,), jnp.int32),
            pltpu.VMEM((VPAD,), jnp.float32),
        ],
    )(idx32, val32)
    soft = soft2d.reshape(2, B, N)

    return (soft, sims, weights)
